# baseline (speedup 1.0000x reference)
import functools

import jax
import jax.numpy as jnp
import numpy as np
from jax import lax
from jax.experimental import pallas as pl
from jax.experimental.pallas import tpu as pltpu
from jax.experimental.pallas import tpu_sc as plsc

_NEG = np.float32(-1e9)
_I = jnp.int32
_N = 32768
_PER = 2048
_SH = (23, 15, 7, 0)
_MESH = plsc.VectorSubcoreMesh(core_axis_name="c", subcore_axis_name="s")


def _stats_body(s_ref, l_ref, cs_ref, ls_ref):
    y = jax.nn.sigmoid(s_ref[...])
    cs_ref[0, 0, :] = jnp.sum(y, axis=0)
    ls_ref[0, 0, :] = jnp.sum(l_ref[...], axis=0)


def _sc_select_body(cs_hbm, ls_hbm, k_hbm, keep_hbm,
                    keys_v, ls_v, hist_v, mh_v, sbuf_v, gh_v, out_v, tmp_v,
                    rb_v, cnt_v, sh_hist, sh_gh, sh_cnt):
    c = lax.axis_index("c")
    s = lax.axis_index("s")

    @pl.when(c == 0)
    def _work():
        base0 = s * _PER
        pltpu.sync_copy(cs_hbm.at[pl.ds(base0, _PER)], keys_v)
        pltpu.sync_copy(ls_hbm.at[pl.ds(base0, _PER)], ls_v)
        pltpu.sync_copy(k_hbm, tmp_v)
        k_rem = tmp_v[...]
        lane = lax.iota(_I, 16)
        ones = jnp.ones((16,), _I)
        zero16 = jnp.zeros((16,), _I)
        svec = jnp.broadcast_to(s, (16,)).astype(_I)

        prefix = zero16
        for m, sh in enumerate(_SH):
            nbits = 7 if m == 3 else 8
            bmask = _I((1 << nbits) - 1)

            def zb(i, _):
                hist_v[pl.ds(i * 16, 16)] = zero16
                return 0
            lax.fori_loop(0, 256, zb, 0, unroll=16)

            def scat(i, _):
                kv = keys_v[pl.ds(i * 16, 16)]
                b = lax.shift_right_logical(kv, _I(sh)) & bmask
                addr = lane * _I(256) + b
                if m == 0:
                    plsc.addupdate_scatter(hist_v, [addr], ones)
                else:
                    valid = lax.shift_right_logical(
                        kv, _I(sh + nbits)) == prefix
                    plsc.addupdate_scatter(hist_v, [addr], ones, mask=valid)
                return 0
            lax.fori_loop(0, _PER // 16, scat, 0, unroll=8)

            for g in range(16):
                acc = hist_v[pl.ds(g * 16, 16)]
                for l in range(1, 16):
                    acc = acc + hist_v[pl.ds(l * 256 + g * 16, 16)]
                mh_v[pl.ds(g * 16, 16)] = acc

            plsc.subcore_barrier()
            pltpu.sync_copy(mh_v, sh_hist.at[s])
            plsc.subcore_barrier()

            @pl.when(s == 0)
            def _merge():
                pltpu.sync_copy(sh_hist, rb_v)
                for g in range(16):
                    acc = rb_v[0, pl.ds(g * 16, 16)]
                    for r in range(1, 16):
                        acc = acc + rb_v[r, pl.ds(g * 16, 16)]
                    gh_v[pl.ds(g * 16, 16)] = acc
                pltpu.sync_copy(gh_v, sh_gh)
            plsc.subcore_barrier()
            pltpu.sync_copy(sh_gh, gh_v)

            accv = zero16
            bsel = zero16
            found = zero16
            for g in range(15, -1, -1):
                h = gh_v[pl.ds(g * 16, 16)]
                suf = lax.rev(plsc.cumsum(lax.rev(h, (0,))), (0,))
                sv = accv + suf
                sbuf_v[pl.ds(g * 16, 16)] = sv
                accv = accv + lax.reduce_sum(h, (0,))
                cond = sv >= k_rem
                pc = plsc.all_reduce_population_count(cond)
                hit = (pc > 0) & (found == 0)
                bsel = jnp.where(hit, _I(g * 16) + pc - 1, bsel)
                found = jnp.where(pc > 0, ones, found)

            sb = plsc.load_gather(sbuf_v, [bsel])
            cb = plsc.load_gather(gh_v, [bsel])
            k_rem = k_rem - (sb - cb)
            prefix = lax.shift_left(prefix, _I(nbits)) | bsel

        t_key = prefix
        r_ties = k_rem

        my_cnt = plsc.load_gather(mh_v, [bsel])
        tmp_v[...] = my_cnt
        plsc.subcore_barrier()
        pltpu.sync_copy(tmp_v, sh_cnt.at[s])
        plsc.subcore_barrier()
        pltpu.sync_copy(sh_cnt, cnt_v)
        tie_base = zero16
        for r in range(16):
            row = cnt_v[r, pl.ds(0, 16)]
            rvec = jnp.broadcast_to(_I(r), (16,))
            tie_base = tie_base + jnp.where(rvec < svec, row, zero16)

        fone = jnp.ones((16,), jnp.float32)
        fzero = jnp.zeros((16,), jnp.float32)

        def out_step(i, tb):
            sl = pl.ds(i * 16, 16)
            kv = keys_v[sl]
            eq = kv == t_key
            eqi = eq.astype(_I)
            pexc = plsc.cumsum(eqi) - eqi
            keep = (kv > t_key) | (eq & ((tb + pexc) < r_ties)) \
                | (ls_v[sl] > 0)
            out_v[sl] = jnp.where(keep, fone, fzero)
            return tb + plsc.all_reduce_population_count(eq)
        lax.fori_loop(0, _PER // 16, out_step, tie_base, unroll=4)
        pltpu.sync_copy(out_v, keep_hbm.at[pl.ds(base0, _PER)])


@functools.partial(
    pl.kernel, mesh=_MESH,
    out_type=jax.ShapeDtypeStruct((_N,), jnp.float32),
    scratch_types=[
        pltpu.VMEM((_PER,), jnp.int32),
        pltpu.VMEM((_PER,), jnp.int32),
        pltpu.VMEM((4096,), jnp.int32),
        pltpu.VMEM((256,), jnp.int32),
        pltpu.VMEM((256,), jnp.int32),
        pltpu.VMEM((256,), jnp.int32),
        pltpu.VMEM((_PER,), jnp.float32),
        pltpu.VMEM((16,), jnp.int32),
        pltpu.VMEM((16, 256), jnp.int32),
        pltpu.VMEM((16, 16), jnp.int32),
        pltpu.VMEM_SHARED((16, 256), jnp.int32),
        pltpu.VMEM_SHARED((256,), jnp.int32),
        pltpu.VMEM_SHARED((16, 16), jnp.int32),
    ],
    compiler_params=pltpu.CompilerParams(needs_layout_passes=False),
)
def _sc_select(cs_hbm, ls_hbm, k_hbm, keep_hbm, *scratch):
    _sc_select_body(cs_hbm, ls_hbm, k_hbm, keep_hbm, *scratch)


def _mask_body(s_ref, keep_ref, o_ref):
    kp = keep_ref[0]
    o_ref[...] = jnp.where(kp > 0.0, s_ref[...], _NEG)


@jax.jit
def kernel(scores, label, k):
    B, N = scores.shape
    CB = 2048
    nblk = N // CB

    cs3, ls3 = pl.pallas_call(
        _stats_body,
        grid=(nblk,),
        in_specs=[
            pl.BlockSpec((B, CB), lambda j: (0, j)),
            pl.BlockSpec((B, CB), lambda j: (0, j)),
        ],
        out_specs=[
            pl.BlockSpec((1, 1, CB), lambda j: (j, 0, 0)),
            pl.BlockSpec((1, 1, CB), lambda j: (j, 0, 0)),
        ],
        out_shape=[
            jax.ShapeDtypeStruct((nblk, 1, CB), jnp.float32),
            jax.ShapeDtypeStruct((nblk, 1, CB), jnp.int32),
        ],
        compiler_params=pltpu.CompilerParams(
            dimension_semantics=("arbitrary",)),
    )(scores, label)

    keys = lax.bitcast_convert_type(cs3.reshape(N), jnp.int32)
    ls = ls3.reshape(N)
    kvec = jnp.full((16,), k, jnp.int32)

    keep = _sc_select(keys, ls, kvec)
    keep3 = keep.reshape(nblk, 1, CB)

    out = pl.pallas_call(
        _mask_body,
        grid=(nblk,),
        in_specs=[
            pl.BlockSpec((B, CB), lambda j: (0, j)),
            pl.BlockSpec((1, 1, CB), lambda j: (j, 0, 0)),
        ],
        out_specs=pl.BlockSpec((B, CB), lambda j: (0, j)),
        out_shape=jax.ShapeDtypeStruct((B, N), jnp.float32),
        compiler_params=pltpu.CompilerParams(
            dimension_semantics=("arbitrary",)),
    )(scores, keep3)
    return out


# speedup vs baseline: 2.8786x; 2.8786x over previous
import functools

import jax
import jax.numpy as jnp
import numpy as np
from jax import lax
from jax.experimental import pallas as pl
from jax.experimental.pallas import tpu as pltpu

_NEG = np.float32(-1e9)


def _fused_body(k_ref, s_ref, l_ref, o_ref, scr, cs, ls, keep):
    j = pl.program_id(0)

    @pl.when(j < 8)
    def _stats():
        s = s_ref[...]
        scr[:, pl.ds(j * 4096, 4096)] = s
        colsum = jnp.sum(jax.nn.sigmoid(s), axis=0)
        lsum = jnp.sum(l_ref[...], axis=0)
        for t in range(32):
            row = pl.ds(j * 32 + t, 1)
            cs[row, :] = colsum[t * 128:(t + 1) * 128].reshape(1, 128)
            ls[row, :] = lsum[t * 128:(t + 1) * 128].reshape(1, 128)

    @pl.when(j == 8)
    def _select():
        v = cs[...]
        key = lax.bitcast_convert_type(v, jnp.int32)
        k = k_ref[0]

        def bit_step(i, t):
            cand = t | (jnp.int32(1) << (jnp.int32(30) - i))
            cnt = jnp.sum((key >= cand).astype(jnp.int32))
            return jnp.where(cnt >= k, cand, t)

        t_final = lax.fori_loop(0, 31, bit_step, jnp.int32(0), unroll=True)

        count_gt = jnp.sum((key > t_final).astype(jnp.int32))
        r = (k - count_gt).astype(jnp.float32)

        eq = key == t_final
        ef = eq.astype(jnp.float32)
        li = lax.broadcasted_iota(jnp.int32, (128, 128), 0)
        lj = lax.broadcasted_iota(jnp.int32, (128, 128), 1)
        lt_strict = (li < lj).astype(jnp.float32)
        ri = lax.broadcasted_iota(jnp.int32, (256, 256), 0)
        rj = lax.broadcasted_iota(jnp.int32, (256, 256), 1)
        rt_strict = (ri > rj).astype(jnp.float32)
        ones = jnp.ones((128, 128), jnp.float32)

        pref_row = jnp.dot(ef, lt_strict, preferred_element_type=jnp.float32)
        row_tot = jnp.dot(ef, ones, preferred_element_type=jnp.float32)
        pref_rows = jnp.dot(rt_strict, row_tot,
                            preferred_element_type=jnp.float32)
        prefix = pref_row + pref_rows

        kp = (key > t_final) | (eq & (prefix < r)) | (ls[...] > 0)
        keep[...] = kp.astype(jnp.float32)

    @pl.when(j >= 9)
    def _mask():
        jj = j - 9
        kp = jnp.concatenate(
            [keep[pl.ds(jj * 32 + t, 1), :] for t in range(32)], axis=1)
        s = scr[:, pl.ds(jj * 4096, 4096)]
        o_ref[...] = jnp.where(kp > 0.0, s, _NEG)


@jax.jit
def kernel(scores, label, k):
    B, N = scores.shape
    CB = 4096
    nblk = N // CB
    k_arr = jnp.asarray(k, jnp.int32).reshape(1)

    out = pl.pallas_call(
        _fused_body,
        grid=(2 * nblk + 1,),
        in_specs=[
            pl.BlockSpec(memory_space=pltpu.SMEM),
            pl.BlockSpec((B, CB), lambda j: (0, jnp.minimum(j, 7))),
            pl.BlockSpec((B, CB), lambda j: (0, jnp.minimum(j, 7))),
        ],
        out_specs=pl.BlockSpec(
            (B, CB), lambda j: (0, jnp.maximum(j - 9, 0))),
        out_shape=jax.ShapeDtypeStruct((B, N), jnp.float32),
        scratch_shapes=[
            pltpu.VMEM((B, N), jnp.float32),
            pltpu.VMEM((256, 128), jnp.float32),
            pltpu.VMEM((256, 128), jnp.int32),
            pltpu.VMEM((256, 128), jnp.float32),
        ],
        compiler_params=pltpu.CompilerParams(
            dimension_semantics=("arbitrary",)),
    )(k_arr, scores, label)
    return out


# speedup vs baseline: 3.0834x; 1.0711x over previous
import functools

import jax
import jax.numpy as jnp
import numpy as np
from jax import lax
from jax.experimental import pallas as pl
from jax.experimental.pallas import tpu as pltpu

_NEG = np.float32(-1e9)


def _fused_body(k_ref, s_ref, l_ref, o_ref, scr, cs, ls, keep):
    j = pl.program_id(0)

    @pl.when(j < 4)
    def _stats():
        s = s_ref[...]
        scr[:, pl.ds(j * 8192, 8192)] = s
        colsum = jnp.sum(jax.nn.sigmoid(s), axis=0)
        lsum = jnp.sum(l_ref[...], axis=0)
        for t in range(64):
            row = pl.ds(j * 64 + t, 1)
            cs[row, :] = colsum[t * 128:(t + 1) * 128].reshape(1, 128)
            ls[row, :] = lsum[t * 128:(t + 1) * 128].reshape(1, 128)

    @pl.when(j == 4)
    def _select():
        v = cs[...]
        key = lax.bitcast_convert_type(v, jnp.int32)
        k = k_ref[0]

        def bit_step(i, t):
            cand = t | (jnp.int32(1) << (jnp.int32(30) - i))
            cnt = jnp.sum((key >= cand).astype(jnp.int32))
            return jnp.where(cnt >= k, cand, t)

        t_final = lax.fori_loop(0, 31, bit_step, jnp.int32(0), unroll=True)

        count_gt = jnp.sum((key > t_final).astype(jnp.int32))
        r = (k - count_gt).astype(jnp.float32)

        eq = key == t_final
        ef = eq.astype(jnp.float32)
        li = lax.broadcasted_iota(jnp.int32, (128, 128), 0)
        lj = lax.broadcasted_iota(jnp.int32, (128, 128), 1)
        lt_strict = (li < lj).astype(jnp.float32)
        ri = lax.broadcasted_iota(jnp.int32, (256, 256), 0)
        rj = lax.broadcasted_iota(jnp.int32, (256, 256), 1)
        rt_strict = (ri > rj).astype(jnp.float32)
        ones = jnp.ones((128, 128), jnp.float32)

        pref_row = jnp.dot(ef, lt_strict, preferred_element_type=jnp.float32)
        row_tot = jnp.dot(ef, ones, preferred_element_type=jnp.float32)
        pref_rows = jnp.dot(rt_strict, row_tot,
                            preferred_element_type=jnp.float32)
        prefix = pref_row + pref_rows

        kp = (key > t_final) | (eq & (prefix < r)) | (ls[...] > 0)
        keep[...] = kp.astype(jnp.float32)

    @pl.when(j >= 5)
    def _mask():
        jj = j - 5
        kp = jnp.concatenate(
            [keep[pl.ds(jj * 64 + t, 1), :] for t in range(64)], axis=1)
        s = scr[:, pl.ds(jj * 8192, 8192)]
        o_ref[...] = jnp.where(kp > 0.0, s, _NEG)


@jax.jit
def kernel(scores, label, k):
    B, N = scores.shape
    CB = 8192
    nblk = N // CB
    k_arr = jnp.asarray(k, jnp.int32).reshape(1)

    out = pl.pallas_call(
        _fused_body,
        grid=(2 * nblk + 1,),
        in_specs=[
            pl.BlockSpec(memory_space=pltpu.SMEM),
            pl.BlockSpec((B, CB), lambda j: (0, jnp.minimum(j, 3))),
            pl.BlockSpec((B, CB), lambda j: (0, jnp.minimum(j, 3))),
        ],
        out_specs=pl.BlockSpec(
            (B, CB), lambda j: (0, jnp.maximum(j - 5, 0))),
        out_shape=jax.ShapeDtypeStruct((B, N), jnp.float32),
        scratch_shapes=[
            pltpu.VMEM((B, N), jnp.float32),
            pltpu.VMEM((256, 128), jnp.float32),
            pltpu.VMEM((256, 128), jnp.int32),
            pltpu.VMEM((256, 128), jnp.float32),
        ],
        compiler_params=pltpu.CompilerParams(
            dimension_semantics=("arbitrary",)),
    )(k_arr, scores, label)
    return out


# speedup vs baseline: 3.3267x; 1.0789x over previous
import functools

import jax
import jax.numpy as jnp
import numpy as np
from jax import lax
from jax.experimental import pallas as pl
from jax.experimental.pallas import tpu as pltpu

_NEG = np.float32(-1e9)


def _fused_body(k_ref, s_ref, l_ref, o_ref, scr, cs, ls, keep):
    j = pl.program_id(0)

    @pl.when(j < 4)
    def _stats():
        s = s_ref[...]
        scr[:, pl.ds(j * 8192, 8192)] = s
        colsum = jnp.sum(jax.nn.sigmoid(s), axis=0)
        lsum = jnp.sum(l_ref[...], axis=0)
        for t in range(64):
            row = pl.ds(j * 64 + t, 1)
            cs[row, :] = colsum[t * 128:(t + 1) * 128].reshape(1, 128)
            ls[row, :] = lsum[t * 128:(t + 1) * 128].reshape(1, 128)

    @pl.when(j == 4)
    def _select():
        v = cs[...]
        key = lax.bitcast_convert_type(v, jnp.int32)
        k = k_ref[0]

        def grp_step(i, t):
            shift = jnp.int32(28) - 3 * i
            bits3 = jnp.int32(0)
            for j in range(1, 8):
                cand = t | (jnp.int32(j) << shift)
                cnt = jnp.sum((key >= cand).astype(jnp.int32))
                bits3 = bits3 + (cnt >= k).astype(jnp.int32)
            return t | (bits3 << shift)

        t_final = lax.fori_loop(0, 10, grp_step, jnp.int32(0), unroll=True)
        cand0 = t_final | jnp.int32(1)
        cnt0 = jnp.sum((key >= cand0).astype(jnp.int32))
        t_final = jnp.where(cnt0 >= k, cand0, t_final)

        count_gt = jnp.sum((key > t_final).astype(jnp.int32))
        r = (k - count_gt).astype(jnp.float32)

        eq = key == t_final
        ef = eq.astype(jnp.float32)
        li = lax.broadcasted_iota(jnp.int32, (128, 128), 0)
        lj = lax.broadcasted_iota(jnp.int32, (128, 128), 1)
        lt_strict = (li < lj).astype(jnp.float32)
        ri = lax.broadcasted_iota(jnp.int32, (256, 256), 0)
        rj = lax.broadcasted_iota(jnp.int32, (256, 256), 1)
        rt_strict = (ri > rj).astype(jnp.float32)
        ones = jnp.ones((128, 128), jnp.float32)

        pref_row = jnp.dot(ef, lt_strict, preferred_element_type=jnp.float32)
        row_tot = jnp.dot(ef, ones, preferred_element_type=jnp.float32)
        pref_rows = jnp.dot(rt_strict, row_tot,
                            preferred_element_type=jnp.float32)
        prefix = pref_row + pref_rows

        kp = (key > t_final) | (eq & (prefix < r)) | (ls[...] > 0)
        keep[...] = kp.astype(jnp.float32)

    @pl.when(j >= 5)
    def _mask():
        jj = j - 5
        kp = jnp.concatenate(
            [keep[pl.ds(jj * 64 + t, 1), :] for t in range(64)], axis=1)
        s = scr[:, pl.ds(jj * 8192, 8192)]
        o_ref[...] = jnp.where(kp > 0.0, s, _NEG)


@jax.jit
def kernel(scores, label, k):
    B, N = scores.shape
    CB = 8192
    nblk = N // CB
    k_arr = jnp.asarray(k, jnp.int32).reshape(1)

    out = pl.pallas_call(
        _fused_body,
        grid=(2 * nblk + 1,),
        in_specs=[
            pl.BlockSpec(memory_space=pltpu.SMEM),
            pl.BlockSpec((B, CB), lambda j: (0, jnp.minimum(j, 3))),
            pl.BlockSpec((B, CB), lambda j: (0, jnp.minimum(j, 3))),
        ],
        out_specs=pl.BlockSpec(
            (B, CB), lambda j: (0, jnp.maximum(j - 5, 0))),
        out_shape=jax.ShapeDtypeStruct((B, N), jnp.float32),
        scratch_shapes=[
            pltpu.VMEM((B, N), jnp.float32),
            pltpu.VMEM((256, 128), jnp.float32),
            pltpu.VMEM((256, 128), jnp.int32),
            pltpu.VMEM((256, 128), jnp.float32),
        ],
        compiler_params=pltpu.CompilerParams(
            dimension_semantics=("arbitrary",)),
    )(k_arr, scores, label)
    return out
